# baseline (speedup 1.0000x reference)
import functools, jax, jax.numpy as jnp
from jax.experimental import pallas as pl
from jax.experimental.pallas import tpu as pltpu

NPTS = 131072
BLK = 512
GRID = NPTS // BLK

def _body(xi_ref, yi_ref, xt_ref, yt_ref, o_ref):
    xt = xt_ref[...]
    yt = yt_ref[...]
    xi = xi_ref[...]
    yi = yi_ref[...]

    def gathered(tT, idx):
        r = jnp.broadcast_to((idx & 127)[None, :], (32, BLK))
        b = jnp.broadcast_to((idx >> 7)[None, :], (32, BLK))
        acc = jnp.take_along_axis(tT[:, 0:128], r, axis=1)
        for blk in range(1, 8):
            g = jnp.take_along_axis(tT[:, blk * 128:(blk + 1) * 128], r, axis=1)
            acc = jnp.where(b == blk, g, acc)
        return acc

    o_ref[:, 0:32] = gathered(xt, xi).T
    o_ref[:, 32:64] = gathered(yt, yi).T

def kernel(pixel_coordinates, x_table, y_table):
    coords = pixel_coordinates.reshape(NPTS, 2)
    xi = coords[:, 0]
    yi = coords[:, 1]
    out = pl.pallas_call(
        _body,
        grid=(GRID,),
        in_specs=[
            pl.BlockSpec((BLK,), lambda i: (i,)),
            pl.BlockSpec((BLK,), lambda i: (i,)),
            pl.BlockSpec((32, 1024), lambda i: (0, 0)),
            pl.BlockSpec((32, 1024), lambda i: (0, 0)),
        ],
        out_specs=pl.BlockSpec((BLK, 64), lambda i: (i, 0)),
        out_shape=jax.ShapeDtypeStruct((NPTS, 64), jnp.float32),
    )(xi, yi, x_table.T, y_table.T)
    return out.reshape(16, 8192, 64)


# speedup vs baseline: 1.6318x; 1.6318x over previous
import functools
import jax
import jax.numpy as jnp
from jax import lax
from jax.experimental import pallas as pl
from jax.experimental.pallas import tpu as pltpu, tpu_sc as plsc

BATCH = 16
NUM_POINTS = 8192
TABLE_ROWS = 1024
HALF = 32

NPAIRS = BATCH * NUM_POINTS
NROWS = 2 * NPAIRS
NW = 32
ROWS_PER_W = NROWS // NW
CHUNK = 1024
NCHUNK = ROWS_PER_W // CHUNK
GSIZE = 128
NG = CHUNK // GSIZE

_mesh = plsc.VectorSubcoreMesh(core_axis_name="c", subcore_axis_name="s")


@functools.partial(
    pl.kernel,
    out_type=jax.ShapeDtypeStruct((NROWS, HALF), jnp.float32),
    mesh=_mesh,
    scratch_types=[
        pltpu.VMEM_SHARED((2 * TABLE_ROWS, HALF), jnp.float32),
        pltpu.VMEM((2, NG, GSIZE), jnp.int32),
        pltpu.VMEM((2, CHUNK, HALF), jnp.float32),
        pltpu.SemaphoreType.DMA,
        pltpu.SemaphoreType.DMA,
        pltpu.SemaphoreType.DMA,
        pltpu.SemaphoreType.DMA,
    ],
    compiler_params=pltpu.CompilerParams(use_tc_tiling_on_sc=False),
)
def _sc_gather(coords_hbm, table_hbm, out_hbm, table_sh, idx_v, rows_v,
               gsem0, gsem1, osem0, osem1):
    wid = lax.axis_index("s") * 2 + lax.axis_index("c")
    @pl.when(lax.axis_index("s") == 0)
    def _():
        pltpu.sync_copy(table_hbm, table_sh)

    plsc.subcore_barrier()

    offs = (lax.iota(jnp.int32, 16) & 1) * TABLE_ROWS
    gsem = (gsem0, gsem1)
    osem = (osem0, osem1)

    out_handles = [None, None]
    prev = None
    for g in range(NCHUNK):
        b = g & 1
        row0 = wid * ROWS_PER_W + g * CHUNK
        if out_handles[b] is not None:
            out_handles[b].wait()
            out_handles[b] = None
        crow0 = pl.multiple_of(row0 // GSIZE, 8)
        pltpu.sync_copy(coords_hbm.at[pl.ds(crow0, NG), :], idx_v.at[b])
        for j in range(NG):
            row = idx_v.at[b, j]

            def add_off(i, _):
                sl = pl.ds(i * 16, 16)
                row[sl] = row[sl] + offs
                return 0

            lax.fori_loop(0, GSIZE // 16, add_off, 0)
        gh = [
            pltpu.async_copy(
                table_sh.at[idx_v.at[b, j]],
                rows_v.at[b, pl.ds(j * GSIZE, GSIZE), :],
                gsem[b],
            )
            for j in range(NG)
        ]
        if prev is not None:
            pb, pgh, prow0 = prev
            for c in pgh:
                c.wait()
            out_handles[pb] = pltpu.async_copy(
                rows_v.at[pb], out_hbm.at[pl.ds(prow0, CHUNK), :], osem[pb]
            )
        prev = (b, gh, row0)

    pb, pgh, prow0 = prev
    for c in pgh:
        c.wait()
    out_handles[pb] = pltpu.async_copy(
        rows_v.at[pb], out_hbm.at[pl.ds(prow0, CHUNK), :], osem[pb]
    )
    for h in out_handles:
        if h is not None:
            h.wait()


def kernel(pixel_coordinates, x_table, y_table):
    coords = pixel_coordinates.reshape(NROWS // GSIZE, GSIZE)
    table = jnp.concatenate([x_table, y_table], axis=0)
    out = _sc_gather(coords, table)
    return out.reshape(BATCH, NUM_POINTS, 2 * HALF)
